# Initial kernel scaffold; baseline (speedup 1.0000x reference)
import jax
import jax.numpy as jnp
from jax.experimental import pallas as pl


def kernel(x, edge_index, edge_attr, batch, W0, b0, We, be, Wmsg, bmsg, Wupd, bupd, Wf, bf, Wo1, bo1, Wo2, bo2):
    raise NotImplementedError("write your pallas kernel here")


# speedup vs baseline: 2.1939x; 2.1939x over previous
import functools

import jax
import jax.numpy as jnp
from jax import lax
from jax.experimental import pallas as pl
from jax.experimental.pallas import tpu as pltpu
from jax.experimental.pallas import tpu_sc as plsc

_N = 10000
_E = 320000
_H = 128
_G = 64
_NL = 3
_NP = 10112
_EP = 323584
_NACC = 10208
_BN = 128
_BE = 512
_CH = 128
_EPT = _EP // 16
_NCH = _EPT // _CH
_RPT = _NACC // 16


def _encode_body(x_ref, mk_ref, w0_ref, b0_ref, wh_ref, bm_ref, h_ref, hp_ref):
    xm = x_ref[...] * mk_ref[...]
    h = jnp.maximum(
        jnp.dot(xm, w0_ref[...], preferred_element_type=jnp.float32)
        + b0_ref[...], 0.0)
    h_ref[...] = h
    hp_ref[...] = jnp.dot(h, wh_ref[...],
                          preferred_element_type=jnp.float32) + bm_ref[...]


def _encode(x, mk, w0, b0, wh, bm):
    nb = _NP // _BN
    return pl.pallas_call(
        _encode_body,
        grid=(nb,),
        in_specs=[
            pl.BlockSpec((_BN, 128), lambda i: (i, 0)),
            pl.BlockSpec((_BN, 1), lambda i: (i, 0)),
            pl.BlockSpec((128, 128), lambda i: (0, 0)),
            pl.BlockSpec((1, 128), lambda i: (0, 0)),
            pl.BlockSpec((128, 128), lambda i: (0, 0)),
            pl.BlockSpec((1, 128), lambda i: (0, 0)),
        ],
        out_specs=[
            pl.BlockSpec((_BN, 128), lambda i: (i, 0)),
            pl.BlockSpec((_BN, 128), lambda i: (i, 0)),
        ],
        out_shape=[
            jax.ShapeDtypeStruct((_NP, 128), jnp.float32),
            jax.ShapeDtypeStruct((_NP, 128), jnp.float32),
        ],
    )(x, mk, w0, b0, wh, bm)


def _ep_body(ea_ref, mk_ref, we_ref, be_ref, wc_ref, out_ref):
    eam = ea_ref[...] * mk_ref[...]
    e = jnp.maximum(
        jnp.dot(eam, we_ref[...], preferred_element_type=jnp.float32)
        + be_ref[...], 0.0)
    ep = jnp.dot(e, wc_ref[...], preferred_element_type=jnp.float32)
    for l in range(_NL):
        out_ref[l] = ep[:, l * 128:(l + 1) * 128]


def _ep_all(ea, mk, we, be, wcat):
    nb = _EP // _BE
    return pl.pallas_call(
        _ep_body,
        grid=(nb,),
        in_specs=[
            pl.BlockSpec((_BE, 16), lambda i: (i, 0)),
            pl.BlockSpec((_BE, 1), lambda i: (i, 0)),
            pl.BlockSpec((16, 128), lambda i: (0, 0)),
            pl.BlockSpec((1, 128), lambda i: (0, 0)),
            pl.BlockSpec((128, 384), lambda i: (0, 0)),
        ],
        out_specs=pl.BlockSpec((_NL, _BE, 128), lambda i: (0, i, 0)),
        out_shape=jax.ShapeDtypeStruct((_NL, _EP, 128), jnp.float32),
    )(ea, mk, we, be, wcat)


def _update_body(h_ref, o0_ref, o1_ref, wu_ref, bu_ref, wh_ref, bm_ref,
                 hn_ref, hp_ref):
    s = o0_ref[...]
    nm = o1_ref[...]
    aggr = nm / (s + 1e-16)
    hn = jnp.maximum(
        h_ref[...]
        + jnp.dot(aggr, wu_ref[...], preferred_element_type=jnp.float32)
        + bu_ref[...], 0.0)
    hn_ref[...] = hn
    hp_ref[...] = jnp.dot(hn, wh_ref[...],
                          preferred_element_type=jnp.float32) + bm_ref[...]


def _update(h, o0, o1, wu, bu, wh, bm):
    nb = _NP // _BN
    return pl.pallas_call(
        _update_body,
        grid=(nb,),
        in_specs=[
            pl.BlockSpec((_BN, 128), lambda i: (i, 0)),
            pl.BlockSpec((_BN, 128), lambda i: (i, 0)),
            pl.BlockSpec((_BN, 128), lambda i: (i, 0)),
            pl.BlockSpec((128, 128), lambda i: (0, 0)),
            pl.BlockSpec((1, 128), lambda i: (0, 0)),
            pl.BlockSpec((128, 128), lambda i: (0, 0)),
            pl.BlockSpec((1, 128), lambda i: (0, 0)),
        ],
        out_specs=[
            pl.BlockSpec((_BN, 128), lambda i: (i, 0)),
            pl.BlockSpec((_BN, 128), lambda i: (i, 0)),
        ],
        out_shape=[
            jax.ShapeDtypeStruct((_NP, 128), jnp.float32),
            jax.ShapeDtypeStruct((_NP, 128), jnp.float32),
        ],
    )(h, o0, o1, wu, bu, wh, bm)


def _pool_body(hi_ref, hj_ref, b_ref, wf_ref, bf_ref, w1_ref, b1_ref,
               w2_ref, b2_ref, ri_ref, rj_ref, ai_ref, aj_ref, cnt_ref):
    i = pl.program_id(0)

    @pl.when(i == 0)
    def _():
        ai_ref[...] = jnp.zeros((_G, 128), jnp.float32)
        aj_ref[...] = jnp.zeros((_G, 128), jnp.float32)
        cnt_ref[...] = jnp.zeros((_G, 128), jnp.float32)

    b = b_ref[0, 0].reshape(_BN, 1)
    gid = lax.broadcasted_iota(jnp.int32, (1, _G), 1).astype(jnp.float32)
    onehot = (b == gid).astype(jnp.float32)
    dn = (((0,), (0,)), ((), ()))
    ai_ref[...] += lax.dot_general(onehot, hi_ref[...], dn,
                                   preferred_element_type=jnp.float32)
    aj_ref[...] += lax.dot_general(onehot, hj_ref[...], dn,
                                   preferred_element_type=jnp.float32)
    cnt_ref[...] += lax.dot_general(onehot, jnp.ones((_BN, 128), jnp.float32),
                                    dn, preferred_element_type=jnp.float32)

    @pl.when(i == (_NP // _BN) - 1)
    def _():
        cnt = jnp.maximum(cnt_ref[...], 1.0)
        for acc_ref, out_ref in ((ai_ref, ri_ref), (aj_ref, rj_ref)):
            pooled = acc_ref[...] / cnt
            rs = jnp.dot(pooled, wf_ref[...],
                         preferred_element_type=jnp.float32) + bf_ref[...]
            t = jnp.maximum(
                jnp.dot(rs, w1_ref[...], preferred_element_type=jnp.float32)
                + b1_ref[...], 0.0)
            out_ref[...] = jnp.dot(t, w2_ref[...],
                                   preferred_element_type=jnp.float32) + b2_ref[...]


def _pool_head(hi, hj, bt, wf, bf, w1, b1, w2, b2):
    nb = _NP // _BN
    return pl.pallas_call(
        _pool_body,
        grid=(nb,),
        in_specs=[
            pl.BlockSpec((_BN, 128), lambda i: (i, 0)),
            pl.BlockSpec((_BN, 128), lambda i: (i, 0)),
            pl.BlockSpec((1, 1, _BN), lambda i: (i, 0, 0)),
            pl.BlockSpec((128, 128), lambda i: (0, 0)),
            pl.BlockSpec((1, 128), lambda i: (0, 0)),
            pl.BlockSpec((128, 128), lambda i: (0, 0)),
            pl.BlockSpec((1, 128), lambda i: (0, 0)),
            pl.BlockSpec((128, 64), lambda i: (0, 0)),
            pl.BlockSpec((1, 64), lambda i: (0, 0)),
        ],
        out_specs=[
            pl.BlockSpec((_G, 64), lambda i: (0, 0)),
            pl.BlockSpec((_G, 64), lambda i: (0, 0)),
        ],
        out_shape=[
            jax.ShapeDtypeStruct((_G, 64), jnp.float32),
            jax.ShapeDtypeStruct((_G, 64), jnp.float32),
        ],
        scratch_shapes=[
            pltpu.VMEM((_G, 128), jnp.float32),
            pltpu.VMEM((_G, 128), jnp.float32),
            pltpu.VMEM((_G, 128), jnp.float32),
        ],
    )(hi, hj, bt, wf, bf, w1, b1, w2, b2)


def _make_sc_pass(layer):
    mesh = plsc.VectorSubcoreMesh(core_axis_name="c", subcore_axis_name="s")

    @functools.partial(
        pl.kernel,
        out_type=jax.ShapeDtypeStruct((2, _NACC, 128), jnp.float32),
        mesh=mesh,
        scratch_types=[
            pltpu.VMEM((_CH,), jnp.int32),
            pltpu.VMEM((_CH,), jnp.int32),
            pltpu.VMEM((_CH, 128), jnp.float32),
            pltpu.VMEM((_CH, 128), jnp.float32),
            pltpu.VMEM((_CH, 128), jnp.float32),
            pltpu.VMEM_SHARED((_NACC, 128), jnp.float32),
            pltpu.SemaphoreType.DMA,
        ],
    )
    def sc_pass(hp_hbm, ep_hbm, src_hbm, dst_hbm, out_hbm,
                src_v, dst_v, ep_v, hp_v, o_v, acc, sem):
        c = lax.axis_index("c")
        s = lax.axis_index("s")
        want_num = c == 1

        def zrow(i, carry):
            for jj in range(8):
                o_v[i, pl.ds(jj * 16, 16)] = jnp.zeros((16,), jnp.float32)
            return carry
        lax.fori_loop(0, _CH, zrow, 0)
        r0 = s * 640

        @pl.when(s < 15)
        def _():
            for q in range(5):
                pltpu.sync_copy(o_v, acc.at[pl.ds(r0 + q * _CH, _CH)])

        @pl.when(s == 15)
        def _():
            for q in range(4):
                pltpu.sync_copy(o_v, acc.at[pl.ds(r0 + q * _CH, _CH)])
            pltpu.sync_copy(o_v.at[pl.ds(0, 96)], acc.at[pl.ds(r0 + 512, 96)])

        plsc.subcore_barrier()

        ebase = s * _EPT

        def chunk(j, carry):
            b = ebase + j * _CH
            pltpu.sync_copy(src_hbm.at[pl.ds(b, _CH)], src_v)
            pltpu.sync_copy(dst_hbm.at[pl.ds(b, _CH)], dst_v)
            pltpu.sync_copy(ep_hbm.at[layer, pl.ds(b, _CH)], ep_v)
            pltpu.async_copy(hp_hbm.at[src_v], hp_v, sem).wait()

            def edge(i, cr):
                for jj in range(8):
                    a = hp_v[i, pl.ds(jj * 16, 16)]
                    e = ep_v[i, pl.ds(jj * 16, 16)]
                    m = jnp.maximum(a + e, 0.0)
                    ex = jnp.exp(m)
                    o_v[i, pl.ds(jj * 16, 16)] = jnp.where(want_num, ex * m, ex)
                return cr
            lax.fori_loop(0, _CH, edge, 0)

            pltpu.sync_copy(o_v, acc.at[dst_v], add=True)
            return carry
        lax.fori_loop(0, _NCH, chunk, 0)

        plsc.subcore_barrier()

        @pl.when(s < 15)
        def _():
            pltpu.sync_copy(acc.at[pl.ds(r0, 640)],
                            out_hbm.at[c, pl.ds(r0, 640)])

        @pl.when(s == 15)
        def _():
            pltpu.sync_copy(acc.at[pl.ds(r0, 608)],
                            out_hbm.at[c, pl.ds(r0, 608)])

    return sc_pass


_SC_PASS = [_make_sc_pass(l) for l in range(_NL)]


def _mask_vec(k, n, rate):
    perm = jax.random.permutation(k, n)
    kk = int(rate * n)
    return jnp.ones((n,), jnp.float32).at[perm[:kk]].set(0.0)


def kernel(x, edge_index, edge_attr, batch, W0, b0, We, be, Wmsg, bmsg,
           Wupd, bupd, Wf, bf, Wo1, bo1, Wo2, bo2):
    key = jax.random.key(42)
    k1, k2, k3, k4 = jax.random.split(key, 4)
    mxi = _mask_vec(k1, _N, 0.5)
    mei = _mask_vec(k2, _E, 0.5)
    mxj = _mask_vec(k3, _N, 0.5)
    mej = _mask_vec(k4, _E, 0.5)

    xp = jnp.pad(x, ((0, _NP - _N), (0, 0)))
    eap = jnp.pad(edge_attr, ((0, _EP - _E), (0, 0)))
    srcp = jnp.pad(edge_index[0], (0, _EP - _E))
    dstp = jnp.pad(edge_index[1], (0, _EP - _E), constant_values=_NP)
    btp = jnp.pad(batch.astype(jnp.float32), (0, _NP - _N),
                  constant_values=1e6).reshape(_NP // _BN, 1, _BN)

    b0r = b0.reshape(1, 128)
    ber = be.reshape(1, 128)
    bfr = bf.reshape(1, 128)
    b1r = bo1.reshape(1, 128)
    b2r = bo2.reshape(1, 64)
    wh = [Wmsg[l][:_H] for l in range(_NL)]
    bm = [bmsg[l].reshape(1, 128) for l in range(_NL)]
    wcat = jnp.concatenate([Wmsg[l][_H:] for l in range(_NL)], axis=1)
    bu = [bupd[l].reshape(1, 128) for l in range(_NL)]

    def run_encoder(xmask, emask):
        xm = jnp.pad(xmask, (0, _NP - _N))[:, None]
        em = jnp.pad(emask, (0, _EP - _E))[:, None]
        h, hp = _encode(xp, xm, W0, b0r, wh[0], bm[0])
        ep = _ep_all(eap, em, We, ber, wcat)
        for l in range(_NL):
            o = _SC_PASS[l](hp, ep, srcp, dstp)
            whn = wh[(l + 1) % _NL]
            bmn = bm[(l + 1) % _NL]
            h, hp = _update(h, o[0], o[1], Wupd[l], bu[l], whn, bmn)
        return h

    hi = run_encoder(mxi, mei)
    hj = run_encoder(mxj, mej)

    ri, rj = _pool_head(hi, hj, btp, Wf, bfr, Wo1, b1r, Wo2, b2r)
    return (ri, rj)
